# Initial kernel scaffold; baseline (speedup 1.0000x reference)
import jax
import jax.numpy as jnp
from jax.experimental import pallas as pl


def kernel(x, rbf, idx_i, idx_j, Wk, Wi, bi, Wj, bj, r0_W1, r0_b1, r0_W2, r0_b2, r1_W1, r1_b1, r1_W2, r1_b2, Wd, bd, u):
    raise NotImplementedError("write your pallas kernel here")


# speedup vs baseline: 2.4336x; 2.4336x over previous
import functools

import jax
import jax.numpy as jnp
from jax import lax
from jax.experimental import pallas as pl
from jax.experimental.pallas import tpu as pltpu
from jax.experimental.pallas import tpu_sc as plsc

N = 10000
E = 320000
F = 128
K = 64

NC = 2
NS = 16
NW = NC * NS
EPW = E // NW
B = 80
CHUNKS = EPW // B
NP = 10240
RPT = NP // NS


def _dot_t(a, w):
    return lax.dot_general(a, w, (((1,), (1,)), ((), ())),
                           preferred_element_type=jnp.float32)


def _g_body(rbf_ref, wk_ref, out_ref):
    out_ref[...] = _dot_t(rbf_ref[...], wk_ref[...])


def _edge_matmul(rbf, Wk):
    BE = 4000
    return pl.pallas_call(
        _g_body,
        grid=(E // BE,),
        in_specs=[
            pl.BlockSpec((BE, K), lambda i: (i, 0)),
            pl.BlockSpec((F, K), lambda i: (0, 0)),
        ],
        out_specs=pl.BlockSpec((BE, F), lambda i: (i, 0)),
        out_shape=jax.ShapeDtypeStruct((E, F), jnp.float32),
    )(rbf, Wk)


def _node_body(x_ref, wi_ref, bi_ref, wj_ref, bj_ref, xi_ref, hj_ref):
    xv = x_ref[...]
    xi_ref[...] = _dot_t(xv, wi_ref[...]) + bi_ref[...]
    hj_ref[...] = _dot_t(xv, wj_ref[...]) + bj_ref[...]


def _node_matmuls(x, Wi, bi, Wj, bj):
    return pl.pallas_call(
        _node_body,
        out_shape=(
            jax.ShapeDtypeStruct((N, F), jnp.float32),
            jax.ShapeDtypeStruct((N, F), jnp.float32),
        ),
    )(x, Wi, bi.reshape(1, F), Wj, bj.reshape(1, F))


def _sc_body(g_hbm, hj_hbm, idxi_hbm, idxj_hbm, z_hbm, out_hbm,
             idxi_v, idxj_v, g_v, rows_v, acc, sem):
    c = lax.axis_index("c")
    s = lax.axis_index("s")
    wid = s * NC + c
    ebase = wid * EPW
    nslice = pl.ds(s * RPT, RPT)

    pltpu.sync_copy(z_hbm.at[nslice], acc.at[nslice])
    plsc.subcore_barrier()

    def chunk(k, carry):
        off = ebase + k * B
        pltpu.sync_copy(idxj_hbm.at[pl.ds(off, B)], idxj_v)
        pltpu.sync_copy(idxi_hbm.at[pl.ds(off, B)], idxi_v)
        pltpu.async_copy(hj_hbm.at[idxj_v], rows_v, sem).wait()
        pltpu.sync_copy(g_hbm.at[pl.ds(off, B)], g_v)

        def mul_row(i, carry2):
            for cc in range(F // 16):
                sl = pl.ds(cc * 16, 16)
                rows_v[i, sl] = rows_v[i, sl] * g_v[i, sl]
            return carry2

        lax.fori_loop(0, B, mul_row, 0)
        pltpu.sync_copy(rows_v, acc.at[idxi_v], add=True)
        return carry

    lax.fori_loop(0, CHUNKS, chunk, 0)
    plsc.subcore_barrier()
    pltpu.sync_copy(acc.at[nslice], out_hbm.at[c, nslice])


def _sc_gather_scatter(g, hj, idx_i, idx_j, zeros_nf):
    mesh = plsc.VectorSubcoreMesh(core_axis_name="c", subcore_axis_name="s")
    f = pl.kernel(
        _sc_body,
        out_type=jax.ShapeDtypeStruct((NC, NP, F), jnp.float32),
        mesh=mesh,
        scratch_types=[
            pltpu.VMEM((B,), jnp.int32),
            pltpu.VMEM((B,), jnp.int32),
            pltpu.VMEM((B, F), jnp.float32),
            pltpu.VMEM((B, F), jnp.float32),
            pltpu.VMEM_SHARED((NP, F), jnp.float32),
            pltpu.SemaphoreType.DMA,
        ],
    )
    return f(g, hj, idx_i, idx_j, zeros_nf)


def _fin_body(x_ref, xi_ref, p_ref, w01, b01, w02, b02,
              w11, b11, w12, b12, wd, bd_, u_, out_ref):
    m = xi_ref[...] + p_ref[0, :N, :] + p_ref[1, :N, :]
    t = _dot_t(m, w01[...]) + b01[...]
    m = m + _dot_t(t, w02[...]) + b02[...]
    t = _dot_t(m, w11[...]) + b11[...]
    m = m + _dot_t(t, w12[...]) + b12[...]
    out_ref[...] = u_[...] * x_ref[...] + _dot_t(m, wd[...]) + bd_[...]


def _final(x, xi, parts, r0_W1, r0_b1, r0_W2, r0_b2,
           r1_W1, r1_b1, r1_W2, r1_b2, Wd, bd, u):
    return pl.pallas_call(
        _fin_body,
        out_shape=jax.ShapeDtypeStruct((N, F), jnp.float32),
    )(x, xi, parts, r0_W1, r0_b1.reshape(1, F), r0_W2, r0_b2.reshape(1, F),
      r1_W1, r1_b1.reshape(1, F), r1_W2, r1_b2.reshape(1, F),
      Wd, bd.reshape(1, F), u.reshape(1, F))


def kernel(x, rbf, idx_i, idx_j, Wk, Wi, bi, Wj, bj,
           r0_W1, r0_b1, r0_W2, r0_b2, r1_W1, r1_b1, r1_W2, r1_b2,
           Wd, bd, u):
    xi, hj = _node_matmuls(x, Wi, bi, Wj, bj)
    g = _edge_matmul(rbf, Wk)
    hj_pad = jnp.pad(hj, ((0, NP - N), (0, 0)))
    zeros_nf = jnp.zeros((NP, F), dtype=jnp.float32)
    parts = _sc_gather_scatter(g, hj_pad, idx_i, idx_j, zeros_nf)
    return _final(x, xi, parts, r0_W1, r0_b1, r0_W2, r0_b2,
                  r1_W1, r1_b1, r1_W2, r1_b2, Wd, bd, u)


# speedup vs baseline: 4.0751x; 1.6745x over previous
import functools

import jax
import jax.numpy as jnp
from jax import lax
from jax.experimental import pallas as pl
from jax.experimental.pallas import tpu as pltpu
from jax.experimental.pallas import tpu_sc as plsc

N = 10000
E = 320000
F = 128
K = 64

NC = 2
NS = 16
NW = NC * NS
EPW = E // NW
B = 80
CHUNKS = EPW // B
NP = 10240
RPT = NP // NS


def _dot_t(a, w):
    return lax.dot_general(a, w, (((1,), (1,)), ((), ())),
                           preferred_element_type=jnp.float32)


def _g_body(rbf_ref, wk_ref, out_ref):
    out_ref[...] = _dot_t(rbf_ref[...], wk_ref[...])


def _edge_matmul(rbf, Wk):
    BE = 4000
    return pl.pallas_call(
        _g_body,
        grid=(E // BE,),
        in_specs=[
            pl.BlockSpec((BE, K), lambda i: (i, 0)),
            pl.BlockSpec((F, K), lambda i: (0, 0)),
        ],
        out_specs=pl.BlockSpec((BE, F), lambda i: (i, 0)),
        out_shape=jax.ShapeDtypeStruct((E, F), jnp.float32),
    )(rbf, Wk)


def _node_body(x_ref, wi_ref, bi_ref, wj_ref, bj_ref, xi_ref, hj_ref):
    xv = x_ref[...]
    xi_ref[...] = _dot_t(xv, wi_ref[...]) + bi_ref[...]
    hj_ref[...] = _dot_t(xv, wj_ref[...]) + bj_ref[...]


def _node_matmuls(x, Wi, bi, Wj, bj):
    return pl.pallas_call(
        _node_body,
        out_shape=(
            jax.ShapeDtypeStruct((N, F), jnp.float32),
            jax.ShapeDtypeStruct((N, F), jnp.float32),
        ),
    )(x, Wi, bi.reshape(1, F), Wj, bj.reshape(1, F))


def _sc_body(g_hbm, hj_hbm, idxi_hbm, idxj_hbm, out_hbm,
             ii0, ii1, ii2, ii3, ij0, ij1, ij2, ij3,
             g0, g1, r0, r1, zbuf, acc,
             isem0, isem1, isem2, isem3,
             lsem0, lsem1, gsem0, gsem1, ssem0, ssem1):
    c = lax.axis_index("c")
    s = lax.axis_index("s")
    wid = s * NC + c
    ebase = wid * EPW
    nslice = pl.ds(s * RPT, RPT)
    iibufs = (ii0, ii1, ii2, ii3)
    ijbufs = (ij0, ij1, ij2, ij3)
    gbufs = (g0, g1)
    rbufs = (r0, r1)
    isems = (isem0, isem1, isem2, isem3)
    lsems = (lsem0, lsem1)
    gsems = (gsem0, gsem1)
    ssems = (ssem0, ssem1)

    for i in range(16):
        for cc in range(F // 16):
            zbuf[i, pl.ds(cc * 16, 16)] = jnp.zeros((16,), jnp.float32)

    def zrow(t, carry):
        pltpu.sync_copy(zbuf, acc.at[pl.ds(s * RPT + t * 16, 16)])
        return carry

    lax.fori_loop(0, RPT // 16, zrow, 0)
    plsc.subcore_barrier()

    def start_idx(k, q):
        off = pl.ds(ebase + k * B, B)
        pltpu.async_copy(idxj_hbm.at[off], ijbufs[q], isems[q])
        pltpu.async_copy(idxi_hbm.at[off], iibufs[q], isems[q])

    def wait_idx(q):
        pltpu.make_async_copy(idxj_hbm.at[pl.ds(0, B)], ijbufs[q],
                              isems[q]).wait()
        pltpu.make_async_copy(idxi_hbm.at[pl.ds(0, B)], iibufs[q],
                              isems[q]).wait()

    def start_inputs(k, d, q):
        pltpu.async_copy(hj_hbm.at[ijbufs[q]], rbufs[d], gsems[d])
        pltpu.async_copy(g_hbm.at[pl.ds(ebase + k * B, B)], gbufs[d], lsems[d])

    def wait_inputs(k, d, q):
        pltpu.make_async_copy(hj_hbm.at[ijbufs[q]], rbufs[d],
                              gsems[d]).wait()
        pltpu.make_async_copy(g_hbm.at[pl.ds(ebase + k * B, B)], gbufs[d],
                              lsems[d]).wait()

    def start_scatter(d, q):
        pltpu.async_copy(rbufs[d], acc.at[iibufs[q]], ssems[d], add=True)

    def wait_scatter(d, q):
        pltpu.make_async_copy(rbufs[d], acc.at[iibufs[q]],
                              ssems[d]).wait()

    start_idx(0, 0)
    start_idx(1, 1)
    wait_idx(0)
    start_inputs(0, 0, 0)

    def step(t, carry):
        kk = t * 4
        for b in range(4):
            k = kk + b
            d = b % 2

            @pl.when(k < CHUNKS)
            def _():
                wait_inputs(k, d, b)
                @pl.when(k + 2 < CHUNKS)
                def _():
                    start_idx(k + 2, (b + 2) % 4)

                @pl.when(k >= 1)
                def _():
                    wait_scatter(1 - d, (b + 3) % 4)

                @pl.when(k + 1 < CHUNKS)
                def _():
                    wait_idx((b + 1) % 4)
                    start_inputs(k + 1, 1 - d, (b + 1) % 4)

                @plsc.parallel_loop(0, B, 1, unroll=4)
                def _(i):
                    for cc in range(F // 16):
                        sl = pl.ds(cc * 16, 16)
                        rbufs[d][i, sl] = rbufs[d][i, sl] * gbufs[d][i, sl]

                start_scatter(d, b)

        return carry

    lax.fori_loop(0, (CHUNKS + 3) // 4, step, 0)
    wait_scatter((CHUNKS - 1) % 2, (CHUNKS - 1) % 4)
    plsc.subcore_barrier()
    pltpu.sync_copy(acc.at[nslice], out_hbm.at[c, nslice])


def _sc_gather_scatter(g, hj, idx_i, idx_j):
    mesh = plsc.VectorSubcoreMesh(core_axis_name="c", subcore_axis_name="s")
    f = pl.kernel(
        _sc_body,
        out_type=jax.ShapeDtypeStruct((NC, NP, F), jnp.float32),
        mesh=mesh,
        scratch_types=[
            pltpu.VMEM((B,), jnp.int32),
            pltpu.VMEM((B,), jnp.int32),
            pltpu.VMEM((B,), jnp.int32),
            pltpu.VMEM((B,), jnp.int32),
            pltpu.VMEM((B,), jnp.int32),
            pltpu.VMEM((B,), jnp.int32),
            pltpu.VMEM((B,), jnp.int32),
            pltpu.VMEM((B,), jnp.int32),
            pltpu.VMEM((B, F), jnp.float32),
            pltpu.VMEM((B, F), jnp.float32),
            pltpu.VMEM((B, F), jnp.float32),
            pltpu.VMEM((B, F), jnp.float32),
            pltpu.VMEM((16, F), jnp.float32),
            pltpu.VMEM_SHARED((NP, F), jnp.float32),
            pltpu.SemaphoreType.DMA,
            pltpu.SemaphoreType.DMA,
            pltpu.SemaphoreType.DMA,
            pltpu.SemaphoreType.DMA,
            pltpu.SemaphoreType.DMA,
            pltpu.SemaphoreType.DMA,
            pltpu.SemaphoreType.DMA,
            pltpu.SemaphoreType.DMA,
            pltpu.SemaphoreType.DMA,
            pltpu.SemaphoreType.DMA,
        ],
    )
    return f(g, hj, idx_i, idx_j)


def _fin_body(x_ref, xi_ref, p_ref, w01, b01, w02, b02,
              w11, b11, w12, b12, wd, bd_, u_, out_ref):
    m = xi_ref[...] + p_ref[0, :N, :] + p_ref[1, :N, :]
    t = _dot_t(m, w01[...]) + b01[...]
    m = m + _dot_t(t, w02[...]) + b02[...]
    t = _dot_t(m, w11[...]) + b11[...]
    m = m + _dot_t(t, w12[...]) + b12[...]
    out_ref[...] = u_[...] * x_ref[...] + _dot_t(m, wd[...]) + bd_[...]


def _final(x, xi, parts, r0_W1, r0_b1, r0_W2, r0_b2,
           r1_W1, r1_b1, r1_W2, r1_b2, Wd, bd, u):
    return pl.pallas_call(
        _fin_body,
        out_shape=jax.ShapeDtypeStruct((N, F), jnp.float32),
    )(x, xi, parts, r0_W1, r0_b1.reshape(1, F), r0_W2, r0_b2.reshape(1, F),
      r1_W1, r1_b1.reshape(1, F), r1_W2, r1_b2.reshape(1, F),
      Wd, bd.reshape(1, F), u.reshape(1, F))


def kernel(x, rbf, idx_i, idx_j, Wk, Wi, bi, Wj, bj,
           r0_W1, r0_b1, r0_W2, r0_b2, r1_W1, r1_b1, r1_W2, r1_b2,
           Wd, bd, u):
    xi, hj = _node_matmuls(x, Wi, bi, Wj, bj)
    g = _edge_matmul(rbf, Wk)
    hj_pad = jnp.pad(hj, ((0, NP - N), (0, 0)))
    parts = _sc_gather_scatter(g, hj_pad, idx_i, idx_j)
    return _final(x, xi, parts, r0_W1, r0_b1, r0_W2, r0_b2,
                  r1_W1, r1_b1, r1_W2, r1_b2, Wd, bd, u)
